# baseline (speedup 1.0000x reference)
import functools

import jax
import jax.numpy as jnp
from jax import lax
from jax.experimental import pallas as pl
from jax.experimental.pallas import tpu as pltpu
from jax.experimental.pallas import tpu_sc as plsc

D = 128
KEEP = 10
NC, NS = 2, 16
NW = NC * NS
B = 50000
N_PIECES = 8
PIECE = 200
PER_W = PIECE * N_PIECES
B_PAD = NW * PER_W


def _pool_body(feat_hbm, idx_hbm, out_hbm, *rest):
    idx_bufs = rest[:2 * KEEP]
    acc = rest[2 * KEEP:2 * KEEP + 2]
    sem_i = rest[2 * KEEP + 2:2 * KEEP + 4]
    sem_g = rest[2 * KEEP + 4:2 * KEEP + 6]
    sem_o = rest[2 * KEEP + 6:2 * KEEP + 8]
    wid = lax.axis_index("c") * NS + lax.axis_index("s")
    base = wid * PER_W

    def fire_idx(p):
        b = (p % 2) * KEEP
        return [pltpu.async_copy(
            idx_hbm.at[pl.ds(s * B + base + p * PIECE, PIECE)],
            idx_bufs[b + s], sem_i[p % 2]) for s in range(KEEP)]

    def fire_init(p):
        return pltpu.async_copy(feat_hbm.at[idx_bufs[(p % 2) * KEEP]],
                                acc[p % 2], sem_g[p % 2])

    def fire_adds(p):
        b = (p % 2) * KEEP
        return [pltpu.async_copy(feat_hbm.at[idx_bufs[b + s]], acc[p % 2],
                                 sem_g[p % 2], add=True)
                for s in range(1, KEEP)]

    def fire_out(p):
        return pltpu.async_copy(acc[p % 2],
                                out_hbm.at[pl.ds(base + p * PIECE, PIECE)],
                                sem_o[p % 2])

    def drain(descs):
        for d_ in descs:
            d_.wait()

    idx_d = [None] * (N_PIECES + 1)
    init_d = [None] * (N_PIECES + 1)
    out_d = [None] * N_PIECES

    idx_d[0] = fire_idx(0)
    drain(idx_d[0])
    init_d[0] = fire_init(0)
    idx_d[1] = fire_idx(1)
    for p in range(N_PIECES):
        init_d[p].wait()
        adds = fire_adds(p)
        if p + 1 < N_PIECES:
            drain(idx_d[p + 1])
            if p >= 1:
                out_d[p - 1].wait()
            init_d[p + 1] = fire_init(p + 1)
        drain(adds)
        if p + 2 < N_PIECES:
            idx_d[p + 2] = fire_idx(p + 2)
        out_d[p] = fire_out(p)
    out_d[N_PIECES - 2].wait()
    out_d[N_PIECES - 1].wait()


_pool_call = functools.partial(
    pl.kernel,
    out_type=jax.ShapeDtypeStruct((B_PAD, D), jnp.float32),
    mesh=plsc.VectorSubcoreMesh(core_axis_name="c", subcore_axis_name="s"),
    scratch_types=(
        [pltpu.VMEM((PIECE,), jnp.int32) for _ in range(2 * KEEP)]
        + [pltpu.VMEM((PIECE, D), jnp.float32) for _ in range(2)]
        + [pltpu.SemaphoreType.DMA for _ in range(6)]
    ),
)(_pool_body)


def _mm_body(x_ref, w_ref, o_ref):
    o_ref[...] = jnp.dot(x_ref[...], w_ref[...],
                         preferred_element_type=jnp.float32)


def _matmul(pooled, w_scaled, n_rows, blk):
    return pl.pallas_call(
        _mm_body,
        grid=(n_rows // blk,),
        in_specs=[
            pl.BlockSpec((blk, D), lambda i: (i, 0)),
            pl.BlockSpec((D, D), lambda i: (0, 0)),
        ],
        out_specs=pl.BlockSpec((blk, D), lambda i: (i, 0)),
        out_shape=jax.ShapeDtypeStruct((n_rows, D), jnp.float32),
    )(pooled, w_scaled)


def kernel(feat_table, pool_W, samp_neighs, max_keep):
    n_center = samp_neighs.shape[0] // KEEP
    idx_flat = jnp.pad(samp_neighs, (0, B_PAD - n_center))
    pooled = _pool_call(feat_table, idx_flat)
    w_scaled = pool_W * (1.0 / max_keep)
    return _matmul(pooled, w_scaled, n_center, blk=5000)


# speedup vs baseline: 1.0105x; 1.0105x over previous
import functools

import jax
import jax.numpy as jnp
from jax import lax
from jax.experimental import pallas as pl
from jax.experimental.pallas import tpu as pltpu
from jax.experimental.pallas import tpu_sc as plsc

D = 128
KEEP = 10
NC, NS = 2, 16
NW = NC * NS
B = 50000
N_PIECES = 4
PIECE = 400
PER_W = PIECE * N_PIECES
B_PAD = NW * PER_W


def _pool_body(feat_hbm, idx_hbm, out_hbm, *rest):
    idx_bufs = rest[:2 * KEEP]
    acc = rest[2 * KEEP:2 * KEEP + 2]
    sem_i = rest[2 * KEEP + 2:2 * KEEP + 4]
    sem_g = rest[2 * KEEP + 4:2 * KEEP + 6]
    sem_o = rest[2 * KEEP + 6:2 * KEEP + 8]
    wid = lax.axis_index("c") * NS + lax.axis_index("s")
    base = wid * PER_W

    def fire_idx(p):
        b = (p % 2) * KEEP
        return [pltpu.async_copy(
            idx_hbm.at[pl.ds(s * B + base + p * PIECE, PIECE)],
            idx_bufs[b + s], sem_i[p % 2]) for s in range(KEEP)]

    def fire_init(p):
        return pltpu.async_copy(feat_hbm.at[idx_bufs[(p % 2) * KEEP]],
                                acc[p % 2], sem_g[p % 2])

    def fire_adds(p):
        b = (p % 2) * KEEP
        return [pltpu.async_copy(feat_hbm.at[idx_bufs[b + s]], acc[p % 2],
                                 sem_g[p % 2], add=True)
                for s in range(1, KEEP)]

    def fire_out(p):
        return pltpu.async_copy(acc[p % 2],
                                out_hbm.at[pl.ds(base + p * PIECE, PIECE)],
                                sem_o[p % 2])

    def drain(descs):
        for d_ in descs:
            d_.wait()

    idx_d = [None] * (N_PIECES + 1)
    init_d = [None] * (N_PIECES + 1)
    out_d = [None] * N_PIECES

    idx_d[0] = fire_idx(0)
    drain(idx_d[0])
    init_d[0] = fire_init(0)
    idx_d[1] = fire_idx(1)
    for p in range(N_PIECES):
        init_d[p].wait()
        adds = fire_adds(p)
        if p + 1 < N_PIECES:
            drain(idx_d[p + 1])
            if p >= 1:
                out_d[p - 1].wait()
            init_d[p + 1] = fire_init(p + 1)
        drain(adds)
        if p + 2 < N_PIECES:
            idx_d[p + 2] = fire_idx(p + 2)
        out_d[p] = fire_out(p)
    out_d[N_PIECES - 2].wait()
    out_d[N_PIECES - 1].wait()


_pool_call = functools.partial(
    pl.kernel,
    out_type=jax.ShapeDtypeStruct((B_PAD, D), jnp.float32),
    mesh=plsc.VectorSubcoreMesh(core_axis_name="c", subcore_axis_name="s"),
    scratch_types=(
        [pltpu.VMEM((PIECE,), jnp.int32) for _ in range(2 * KEEP)]
        + [pltpu.VMEM((PIECE, D), jnp.float32) for _ in range(2)]
        + [pltpu.SemaphoreType.DMA for _ in range(6)]
    ),
)(_pool_body)


def _mm_body(x_ref, w_ref, o_ref):
    o_ref[...] = jnp.dot(x_ref[...], w_ref[...],
                         preferred_element_type=jnp.float32)


def _matmul(pooled, w_scaled, n_rows, blk):
    return pl.pallas_call(
        _mm_body,
        grid=(n_rows // blk,),
        in_specs=[
            pl.BlockSpec((blk, D), lambda i: (i, 0)),
            pl.BlockSpec((D, D), lambda i: (0, 0)),
        ],
        out_specs=pl.BlockSpec((blk, D), lambda i: (i, 0)),
        out_shape=jax.ShapeDtypeStruct((n_rows, D), jnp.float32),
    )(pooled, w_scaled)


def kernel(feat_table, pool_W, samp_neighs, max_keep):
    n_center = samp_neighs.shape[0] // KEEP
    idx_flat = jnp.pad(samp_neighs, (0, B_PAD - n_center))
    pooled = _pool_call(feat_table, idx_flat)
    w_scaled = pool_W * (1.0 / max_keep)
    return _matmul(pooled, w_scaled, n_center, blk=5000)


# speedup vs baseline: 1.6646x; 1.6474x over previous
import functools

import jax
import jax.numpy as jnp
from jax import lax
from jax.experimental import pallas as pl
from jax.experimental.pallas import tpu as pltpu
from jax.experimental.pallas import tpu_sc as plsc

D = 128
KEEP = 10
NC, NS = 2, 16
NW = NC * NS
B = 50000
N_PIECES = 4
PIECE = 400
PER_W = PIECE * N_PIECES
B_PAD = NW * PER_W


def _pool_body(feat_hbm, idx_hbm, out_hbm, *rest):
    idx_bufs = rest[:2 * KEEP]
    acc = rest[2 * KEEP:2 * KEEP + 2]
    sem_i = rest[2 * KEEP + 2:2 * KEEP + 4]
    sem_g = rest[2 * KEEP + 4:2 * KEEP + 6]
    sem_o = rest[2 * KEEP + 6:2 * KEEP + 8]
    wid = lax.axis_index("c") * NS + lax.axis_index("s")
    base = wid * PER_W

    def fire_idx(p):
        b = (p % 2) * KEEP
        return [pltpu.async_copy(
            idx_hbm.at[pl.ds(s * B + base + p * PIECE, PIECE)],
            idx_bufs[b + s], sem_i[p % 2]) for s in range(KEEP)]

    def fire_init(p):
        return pltpu.async_copy(feat_hbm.at[idx_bufs[(p % 2) * KEEP]],
                                acc[p % 2], sem_g[p % 2])

    def fire_adds(p):
        b = (p % 2) * KEEP
        return [pltpu.async_copy(feat_hbm.at[idx_bufs[b + s]], acc[p % 2],
                                 sem_g[p % 2], add=True)
                for s in range(1, KEEP)]

    def fire_out(p):
        return pltpu.async_copy(acc[p % 2],
                                out_hbm.at[pl.ds(base + p * PIECE, PIECE)],
                                sem_o[p % 2])

    def drain(descs):
        for d_ in descs:
            d_.wait()

    idx_d = [None] * (N_PIECES + 1)
    init_d = [None] * (N_PIECES + 1)
    out_d = [None] * N_PIECES

    idx_d[0] = fire_idx(0)
    drain(idx_d[0])
    init_d[0] = fire_init(0)
    idx_d[1] = fire_idx(1)
    for p in range(N_PIECES):
        init_d[p].wait()
        adds = fire_adds(p)
        if p + 1 < N_PIECES:
            drain(idx_d[p + 1])
            if p >= 1:
                out_d[p - 1].wait()
            init_d[p + 1] = fire_init(p + 1)
        drain(adds)
        if p + 2 < N_PIECES:
            idx_d[p + 2] = fire_idx(p + 2)
        out_d[p] = fire_out(p)
    out_d[N_PIECES - 2].wait()
    out_d[N_PIECES - 1].wait()


_pool_call = functools.partial(
    pl.kernel,
    out_type=jax.ShapeDtypeStruct((B_PAD, D), jnp.float32),
    mesh=plsc.VectorSubcoreMesh(core_axis_name="c", subcore_axis_name="s"),
    scratch_types=(
        [pltpu.VMEM((PIECE,), jnp.int32) for _ in range(2 * KEEP)]
        + [pltpu.VMEM((PIECE, D), jnp.float32) for _ in range(2)]
        + [pltpu.SemaphoreType.DMA for _ in range(6)]
    ),
)(_pool_body)


def _mm_body(x_ref, w_ref, o_ref):
    o_ref[...] = jnp.dot(x_ref[...], w_ref[...],
                         preferred_element_type=jnp.float32)


def _matmul(pooled, w_scaled, n_rows, blk):
    return pl.pallas_call(
        _mm_body,
        grid=(n_rows // blk,),
        in_specs=[
            pl.BlockSpec((blk, D), lambda i: (i, 0)),
            pl.BlockSpec((D, D), lambda i: (0, 0)),
        ],
        out_specs=pl.BlockSpec((blk, D), lambda i: (i, 0)),
        out_shape=jax.ShapeDtypeStruct((n_rows, D), jnp.float32),
    )(pooled, w_scaled)


def kernel(feat_table, pool_W, samp_neighs, max_keep):
    n_center = samp_neighs.shape[0] // KEEP
    n_pad = B_PAD - n_center
    pad_idx = (jnp.arange(n_pad, dtype=jnp.int32) * 83) % feat_table.shape[0]
    idx_flat = jnp.concatenate([samp_neighs, pad_idx])
    pooled = _pool_call(feat_table, idx_flat)
    w_scaled = pool_W * (1.0 / max_keep)
    return _matmul(pooled, w_scaled, n_center, blk=5000)


# speedup vs baseline: 1.6663x; 1.0010x over previous
import functools

import jax
import jax.numpy as jnp
from jax import lax
from jax.experimental import pallas as pl
from jax.experimental.pallas import tpu as pltpu
from jax.experimental.pallas import tpu_sc as plsc

D = 128
KEEP = 10
NC, NS = 2, 16
NW = NC * NS
B = 50000
N_PIECES = 4
PIECE = 392
PER_W = PIECE * N_PIECES
B_PAD = NW * PER_W


def _pool_body(feat_hbm, idx_hbm, out_hbm, *rest):
    idx_bufs = rest[:2 * KEEP]
    acc = rest[2 * KEEP:2 * KEEP + 2]
    sem_i = rest[2 * KEEP + 2:2 * KEEP + 4]
    sem_g = rest[2 * KEEP + 4:2 * KEEP + 6]
    sem_o = rest[2 * KEEP + 6:2 * KEEP + 8]
    wid = lax.axis_index("c") * NS + lax.axis_index("s")
    base = wid * PER_W

    def fire_idx(p):
        b = (p % 2) * KEEP
        return [pltpu.async_copy(
            idx_hbm.at[pl.ds(s * B + base + p * PIECE, PIECE)],
            idx_bufs[b + s], sem_i[p % 2]) for s in range(KEEP)]

    def fire_init(p):
        return pltpu.async_copy(feat_hbm.at[idx_bufs[(p % 2) * KEEP]],
                                acc[p % 2], sem_g[p % 2])

    def fire_adds(p):
        b = (p % 2) * KEEP
        return [pltpu.async_copy(feat_hbm.at[idx_bufs[b + s]], acc[p % 2],
                                 sem_g[p % 2], add=True)
                for s in range(1, KEEP)]

    def fire_out(p):
        return pltpu.async_copy(acc[p % 2],
                                out_hbm.at[pl.ds(base + p * PIECE, PIECE)],
                                sem_o[p % 2])

    def drain(descs):
        for d_ in descs:
            d_.wait()

    idx_d = [None] * (N_PIECES + 1)
    init_d = [None] * (N_PIECES + 1)
    out_d = [None] * N_PIECES

    idx_d[0] = fire_idx(0)
    drain(idx_d[0])
    init_d[0] = fire_init(0)
    idx_d[1] = fire_idx(1)
    for p in range(N_PIECES):
        init_d[p].wait()
        adds = fire_adds(p)
        if p + 1 < N_PIECES:
            drain(idx_d[p + 1])
            if p >= 1:
                out_d[p - 1].wait()
            init_d[p + 1] = fire_init(p + 1)
        drain(adds)
        if p + 2 < N_PIECES:
            idx_d[p + 2] = fire_idx(p + 2)
        out_d[p] = fire_out(p)
    out_d[N_PIECES - 2].wait()
    out_d[N_PIECES - 1].wait()


_pool_call = functools.partial(
    pl.kernel,
    out_type=jax.ShapeDtypeStruct((B_PAD, D), jnp.float32),
    mesh=plsc.VectorSubcoreMesh(core_axis_name="c", subcore_axis_name="s"),
    scratch_types=(
        [pltpu.VMEM((PIECE,), jnp.int32) for _ in range(2 * KEEP)]
        + [pltpu.VMEM((PIECE, D), jnp.float32) for _ in range(2)]
        + [pltpu.SemaphoreType.DMA for _ in range(6)]
    ),
)(_pool_body)


def _mm_body(x_ref, w_ref, o_ref):
    o_ref[...] = jnp.dot(x_ref[...], w_ref[...],
                         preferred_element_type=jnp.float32)


def _matmul(pooled, w_scaled, n_rows, blk):
    return pl.pallas_call(
        _mm_body,
        grid=(n_rows // blk,),
        in_specs=[
            pl.BlockSpec((blk, D), lambda i: (i, 0)),
            pl.BlockSpec((D, D), lambda i: (0, 0)),
        ],
        out_specs=pl.BlockSpec((blk, D), lambda i: (i, 0)),
        out_shape=jax.ShapeDtypeStruct((n_rows, D), jnp.float32),
    )(pooled, w_scaled)


def kernel(feat_table, pool_W, samp_neighs, max_keep):
    n_center = samp_neighs.shape[0] // KEEP
    n_pad = B_PAD - n_center
    pad_idx = (jnp.arange(n_pad, dtype=jnp.int32) * 83) % feat_table.shape[0]
    idx_flat = jnp.concatenate([samp_neighs, pad_idx])
    pooled = _pool_call(feat_table, idx_flat)
    w_scaled = pool_W * (1.0 / max_keep)
    return _matmul(pooled, w_scaled, n_center, blk=5000)


# speedup vs baseline: 1.7006x; 1.0206x over previous
import functools

import jax
import jax.numpy as jnp
from jax import lax
from jax.experimental import pallas as pl
from jax.experimental.pallas import tpu as pltpu
from jax.experimental.pallas import tpu_sc as plsc

D = 128
KEEP = 10
NC, NS = 2, 16
NW = NC * NS
B = 50000
N_PIECES = 4
PIECE = 392
PER_W = PIECE * N_PIECES
B_PAD = NW * PER_W


def _pool_body(feat_hbm, idx_hbm, out_hbm, *rest):
    idx_bufs = rest[:2 * KEEP]
    acc = rest[2 * KEEP:2 * KEEP + 2]
    sem_i = rest[2 * KEEP + 2:2 * KEEP + 4]
    sem_g = rest[2 * KEEP + 4:2 * KEEP + 6]
    sem_o = rest[2 * KEEP + 6:2 * KEEP + 8]
    wid = lax.axis_index("c") * NS + lax.axis_index("s")
    base = wid * PER_W

    def fire_idx(p):
        b = (p % 2) * KEEP
        return [pltpu.async_copy(
            idx_hbm.at[pl.ds(s * B + base + p * PIECE, PIECE)],
            idx_bufs[b + s], sem_i[p % 2]) for s in range(KEEP)]

    def fire_init(p):
        return pltpu.async_copy(feat_hbm.at[idx_bufs[(p % 2) * KEEP]],
                                acc[p % 2], sem_g[p % 2])

    def fire_adds(p):
        b = (p % 2) * KEEP
        return [pltpu.async_copy(feat_hbm.at[idx_bufs[b + s]], acc[p % 2],
                                 sem_g[p % 2], add=True)
                for s in range(1, KEEP)]

    def fire_out(p):
        return pltpu.async_copy(acc[p % 2],
                                out_hbm.at[pl.ds(base + p * PIECE, PIECE)],
                                sem_o[p % 2])

    def drain(descs):
        for d_ in descs:
            d_.wait()

    idx_d = [None] * (N_PIECES + 1)
    init_d = [None] * (N_PIECES + 1)
    out_d = [None] * N_PIECES

    idx_d[0] = fire_idx(0)
    drain(idx_d[0])
    init_d[0] = fire_init(0)
    idx_d[1] = fire_idx(1)
    for p in range(N_PIECES):
        init_d[p].wait()
        adds = fire_adds(p)
        if p + 1 < N_PIECES:
            drain(idx_d[p + 1])
            if p >= 1:
                out_d[p - 1].wait()
            init_d[p + 1] = fire_init(p + 1)
        drain(adds)
        if p + 2 < N_PIECES:
            idx_d[p + 2] = fire_idx(p + 2)
        out_d[p] = fire_out(p)
    out_d[N_PIECES - 2].wait()
    out_d[N_PIECES - 1].wait()


_pool_call = functools.partial(
    pl.kernel,
    out_type=jax.ShapeDtypeStruct((B_PAD, D), jnp.float32),
    mesh=plsc.VectorSubcoreMesh(core_axis_name="c", subcore_axis_name="s"),
    scratch_types=(
        [pltpu.VMEM((PIECE,), jnp.int32) for _ in range(2 * KEEP)]
        + [pltpu.VMEM((PIECE, D), jnp.float32) for _ in range(2)]
        + [pltpu.SemaphoreType.DMA for _ in range(6)]
    ),
)(_pool_body)


def _mm_body(x_ref, w_ref, o_ref):
    o_ref[...] = jnp.dot(x_ref[...], w_ref[...],
                         preferred_element_type=jnp.float32)


def _matmul(pooled, w_scaled, n_rows, blk):
    return pl.pallas_call(
        _mm_body,
        grid=(n_rows // blk,),
        in_specs=[
            pl.BlockSpec((blk, D), lambda i: (i, 0)),
            pl.BlockSpec((D, D), lambda i: (0, 0)),
        ],
        out_specs=pl.BlockSpec((blk, D), lambda i: (i, 0)),
        out_shape=jax.ShapeDtypeStruct((n_rows, D), jnp.float32),
    )(pooled, w_scaled)


def kernel(feat_table, pool_W, samp_neighs, max_keep):
    n_center = samp_neighs.shape[0] // KEEP
    n_pad = B_PAD - n_center
    pad_idx = (jnp.arange(n_pad, dtype=jnp.int32) * 83) % feat_table.shape[0]
    idx_flat = jnp.concatenate([samp_neighs, pad_idx])
    pooled = _pool_call(feat_table, idx_flat)
    w_scaled = pool_W * (1.0 / max_keep)
    return _matmul(pooled, w_scaled, n_center, blk=10000)


# speedup vs baseline: 1.7144x; 1.0081x over previous
import functools

import jax
import jax.numpy as jnp
from jax import lax
from jax.experimental import pallas as pl
from jax.experimental.pallas import tpu as pltpu
from jax.experimental.pallas import tpu_sc as plsc

D = 128
KEEP = 10
NC, NS = 2, 16
NW = NC * NS
B = 50000
N_PIECES = 4
PIECE = 392
PER_W = PIECE * N_PIECES
B_PAD = NW * PER_W


def _pool_body(feat_hbm, idx_hbm, out_hbm, *rest):
    idx_bufs = rest[:2 * KEEP]
    acc = rest[2 * KEEP:2 * KEEP + 2]
    sem_i = rest[2 * KEEP + 2:2 * KEEP + 4]
    sem_g = rest[2 * KEEP + 4:2 * KEEP + 6]
    sem_o = rest[2 * KEEP + 6:2 * KEEP + 8]
    wid = lax.axis_index("c") * NS + lax.axis_index("s")
    base = wid * PER_W

    def fire_idx(p):
        b = (p % 2) * KEEP
        return [pltpu.async_copy(
            idx_hbm.at[pl.ds(s * B + base + p * PIECE, PIECE)],
            idx_bufs[b + s], sem_i[p % 2]) for s in range(KEEP)]

    def fire_init(p):
        return pltpu.async_copy(feat_hbm.at[idx_bufs[(p % 2) * KEEP]],
                                acc[p % 2], sem_g[p % 2])

    def fire_adds(p):
        b = (p % 2) * KEEP
        return [pltpu.async_copy(feat_hbm.at[idx_bufs[b + s]], acc[p % 2],
                                 sem_g[p % 2], add=True)
                for s in range(1, KEEP)]

    def fire_out(p):
        return pltpu.async_copy(acc[p % 2],
                                out_hbm.at[pl.ds(base + p * PIECE, PIECE)],
                                sem_o[p % 2])

    def drain(descs):
        for d_ in descs:
            d_.wait()

    idx_d = [None] * (N_PIECES + 1)
    init_d = [None] * (N_PIECES + 1)
    out_d = [None] * N_PIECES

    idx_d[0] = fire_idx(0)
    drain(idx_d[0])
    init_d[0] = fire_init(0)
    idx_d[1] = fire_idx(1)
    for p in range(N_PIECES):
        init_d[p].wait()
        adds = fire_adds(p)
        if p + 1 < N_PIECES:
            drain(idx_d[p + 1])
            if p >= 1:
                out_d[p - 1].wait()
            init_d[p + 1] = fire_init(p + 1)
        drain(adds)
        if p + 2 < N_PIECES:
            idx_d[p + 2] = fire_idx(p + 2)
        out_d[p] = fire_out(p)
    out_d[N_PIECES - 2].wait()
    out_d[N_PIECES - 1].wait()


_pool_call = functools.partial(
    pl.kernel,
    out_type=jax.ShapeDtypeStruct((B_PAD, D), jnp.float32),
    mesh=plsc.VectorSubcoreMesh(core_axis_name="c", subcore_axis_name="s"),
    scratch_types=(
        [pltpu.VMEM((PIECE,), jnp.int32) for _ in range(2 * KEEP)]
        + [pltpu.VMEM((PIECE, D), jnp.float32) for _ in range(2)]
        + [pltpu.SemaphoreType.DMA for _ in range(6)]
    ),
)(_pool_body)


def _mm_body(x_ref, w_ref, o_ref):
    o_ref[...] = jnp.dot(x_ref[...], w_ref[...],
                         preferred_element_type=jnp.float32)


def _matmul(pooled, w_scaled, n_rows, blk):
    return pl.pallas_call(
        _mm_body,
        grid=(n_rows // blk,),
        in_specs=[
            pl.BlockSpec((blk, D), lambda i: (i, 0)),
            pl.BlockSpec((D, D), lambda i: (0, 0)),
        ],
        out_specs=pl.BlockSpec((blk, D), lambda i: (i, 0)),
        out_shape=jax.ShapeDtypeStruct((n_rows, D), jnp.float32),
    )(pooled, w_scaled)


def kernel(feat_table, pool_W, samp_neighs, max_keep):
    n_center = samp_neighs.shape[0] // KEEP
    n_pad = B_PAD - n_center
    pad_idx = (jnp.arange(n_pad, dtype=jnp.int32) * 83) % feat_table.shape[0]
    idx_flat = jnp.concatenate([samp_neighs, pad_idx])
    pooled = _pool_call(feat_table, idx_flat)
    w_scaled = pool_W * (1.0 / max_keep)
    return _matmul(pooled, w_scaled, n_center, blk=25000)
